# baseline (speedup 1.0000x reference)
import functools

import jax
import jax.numpy as jnp
from jax import lax
from jax.experimental import pallas as pl
from jax.experimental.pallas import tpu as pltpu
from jax.experimental.pallas import tpu_sc as plsc

D = 32
L = 16
NC = 2
NS = 16
NW = NC * NS
ROWS = 1000000
P = 999936
TAIL = ROWS - P


def _stage_body(in_ref, out_ref, sem):
    cps = [
        pltpu.make_async_copy(
            in_ref.at[d, pl.ds(0, P)], out_ref.at[pl.ds(d * P, P)], sem)
        for d in range(D)
    ]
    for cp in cps:
        cp.start()
    for cp in cps:
        cp.wait()


def _stage(tableT):
    return pl.pallas_call(
        _stage_body,
        in_specs=[pl.BlockSpec(memory_space=pl.ANY)],
        out_specs=pl.BlockSpec(memory_space=pl.ANY),
        out_shape=jax.ShapeDtypeStruct((D * P,), jnp.float32),
        scratch_shapes=[pltpu.SemaphoreType.DMA],
    )(tableT)


def _gmf_body(user_hbm, item_hbm, utf_hbm, itf_hbm, utl_hbm, itl_hbm,
              wb_hbm, bb_hbm, out_hbm,
              uidx, iidx, uoff, ioff, urf, irf, utlv, itlv,
              outv, wv, bv, sem_u, sem_i, bpw):
    wid = lax.axis_index("s") * NC + lax.axis_index("c")
    base = wid * bpw

    pltpu.sync_copy(user_hbm.at[pl.ds(base, bpw)], uidx)
    pltpu.sync_copy(item_hbm.at[pl.ds(base, bpw)], iidx)
    pltpu.sync_copy(wb_hbm, wv)
    pltpu.sync_copy(bb_hbm, bv)
    pltpu.sync_copy(utl_hbm, utlv)
    pltpu.sync_copy(itl_hbm, itlv)

    cvecs = [jnp.full((L,), d * P, jnp.int32) for d in range(D)]
    tvecs = [jnp.full((L,), d * TAIL, jnp.int32) for d in range(D)]
    pvec = jnp.full((L,), P, jnp.int32)
    zvec = jnp.full((L,), 0, jnp.int32)

    def offsets(g, _):
        j0 = g * L
        rv_u = uidx[pl.ds(j0, L)]
        rv_i = iidx[pl.ds(j0, L)]
        cu = jnp.where(rv_u < pvec, rv_u, zvec)
        ci = jnp.where(rv_i < pvec, rv_i, zvec)
        for d in range(D):
            uoff[pl.ds(d * bpw + j0, L)] = cu + cvecs[d]
            ioff[pl.ds(d * bpw + j0, L)] = ci + cvecs[d]
        return 0

    lax.fori_loop(0, bpw // L, offsets, 0)

    cp_u = pltpu.async_copy(utf_hbm.at[uoff], urf, sem_u)
    cp_i = pltpu.async_copy(itf_hbm.at[ioff], irf, sem_i)
    cp_u.wait()
    cp_i.wait()

    wvecs = [wv[pl.ds(d * L, L)] for d in range(D)]
    bvec = bv[...]

    def group(g, _):
        j0 = g * L
        rv_u = uidx[pl.ds(j0, L)]
        rv_i = iidx[pl.ds(j0, L)]
        mu = rv_u < pvec
        mi = rv_i < pvec
        tu = jnp.where(mu, zvec, rv_u - pvec)
        ti = jnp.where(mi, zvec, rv_i - pvec)
        acc = bvec
        for d in range(D):
            sl = pl.ds(d * bpw + j0, L)
            ut = plsc.load_gather(utlv, [tu + tvecs[d]])
            it = plsc.load_gather(itlv, [ti + tvecs[d]])
            uc = jnp.where(mu, urf[sl], ut)
            ic = jnp.where(mi, irf[sl], it)
            acc = acc + uc * ic * wvecs[d]
        outv[pl.ds(j0, L)] = acc
        return 0

    lax.fori_loop(0, bpw // L, group, 0)

    pltpu.sync_copy(outv, out_hbm.at[pl.ds(base, bpw)])


def kernel(user, item, user_table, item_table, W, b):
    batch = user.shape[0]
    bpw = batch // NW
    mesh = plsc.VectorSubcoreMesh(core_axis_name="c", subcore_axis_name="s")

    wb = jnp.broadcast_to(W.reshape(D, 1), (D, L)).reshape(D * L)
    wb = wb.astype(jnp.float32)
    bb = jnp.broadcast_to(b.reshape(1), (L,)).astype(jnp.float32)

    uflat = _stage(user_table.T)
    iflat = _stage(item_table.T)
    utail = user_table[P:, :].T.reshape(-1)
    itail = item_table[P:, :].T.reshape(-1)

    k = functools.partial(
        pl.kernel,
        mesh=mesh,
        out_type=jax.ShapeDtypeStruct((batch,), jnp.float32),
        scratch_types=[
            pltpu.VMEM((bpw,), jnp.int32),
            pltpu.VMEM((bpw,), jnp.int32),
            pltpu.VMEM((D * bpw,), jnp.int32),
            pltpu.VMEM((D * bpw,), jnp.int32),
            pltpu.VMEM((D * bpw,), jnp.float32),
            pltpu.VMEM((D * bpw,), jnp.float32),
            pltpu.VMEM((D * TAIL,), jnp.float32),
            pltpu.VMEM((D * TAIL,), jnp.float32),
            pltpu.VMEM((bpw,), jnp.float32),
            pltpu.VMEM((D * L,), jnp.float32),
            pltpu.VMEM((L,), jnp.float32),
            pltpu.SemaphoreType.DMA,
            pltpu.SemaphoreType.DMA,
        ],
        compiler_params=pltpu.CompilerParams(
            needs_layout_passes=False, use_tc_tiling_on_sc=False),
    )(functools.partial(_gmf_body, bpw=bpw))

    return k(user.astype(jnp.int32), item.astype(jnp.int32),
             uflat, iflat, utail, itail, wb, bb)


# speedup vs baseline: 11.0483x; 11.0483x over previous
import functools

import jax
import jax.numpy as jnp
from jax import lax
from jax.experimental import pallas as pl
from jax.experimental.pallas import tpu as pltpu
from jax.experimental.pallas import tpu_sc as plsc

D = 32
L = 16
NC = 2
NS = 16
NW = NC * NS
ROWS = 1000000
PK = 4
PW = PK * D
HB = 2

SCW = 4096
SRW = SCW // PK
NP = pl.cdiv(ROWS, SCW)
SR = NP * SRW


def _stage_body(in_ref, out_ref, xs_ref):
    xs_ref[...] = in_ref[...].T
    for s in range(PK):
        out_ref[:, s * D:(s + 1) * D] = xs_ref[s::PK, :]


def _stage(tableT):
    return pl.pallas_call(
        _stage_body,
        grid=(NP,),
        in_specs=[pl.BlockSpec((D, SCW), lambda p: (0, p))],
        out_specs=pl.BlockSpec((SRW, PW), lambda p: (p, 0)),
        out_shape=jax.ShapeDtypeStruct((SR, PW), jnp.float32),
        scratch_shapes=[pltpu.VMEM((SCW, D), jnp.float32)],
    )(tableT)


def _gmf_body(user_hbm, item_hbm, ut_hbm, it_hbm, wb_hbm, bb_hbm, out_hbm,
              uidx, iidx, ublk, urb, irb, outv, wv, bv, sem_u, sem_i, bpw):
    wid = lax.axis_index("s") * NC + lax.axis_index("c")
    base = wid * bpw
    ch = bpw // HB

    pltpu.sync_copy(user_hbm.at[pl.ds(base, bpw)], uidx)
    pltpu.sync_copy(item_hbm.at[pl.ds(base, bpw)], iidx)
    pltpu.sync_copy(wb_hbm, wv)
    pltpu.sync_copy(bb_hbm, bv)

    def blkidx(g, _):
        j0 = g * L
        ublk[pl.ds(j0, L)] = uidx[pl.ds(j0, L)] >> 2
        return 0

    lax.fori_loop(0, bpw // L, blkidx, 0)

    wvecs = [wv[pl.ds(d * L, L)] for d in range(D)]
    bvec = bv[...]
    lane = lax.iota(jnp.int32, L)
    subm = jnp.full((L,), PK - 1, jnp.int32)
    d32 = jnp.full((L,), D, jnp.int32)

    for h in range(HB):
        cp_u = pltpu.async_copy(
            ut_hbm.at[ublk.at[pl.ds(h * ch, ch)]], urb, sem_u)
        cp_i = pltpu.async_copy(
            it_hbm.at[iidx.at[pl.ds(h * ch, ch)]], irb, sem_i)
        cp_u.wait()
        cp_i.wait()

        def group(g, _):
            j0 = h * ch + g * L
            rows = g * L + lane
            cu = (uidx[pl.ds(j0, L)] & subm) * d32
            acc = bvec
            for d in range(D):
                cd = jnp.full((L,), d, jnp.int32)
                uc = plsc.load_gather(urb, [rows, cu + cd])
                ic = plsc.load_gather(irb, [rows, cd])
                acc = acc + uc * ic * wvecs[d]
            outv[pl.ds(j0, L)] = acc
            return 0

        lax.fori_loop(0, ch // L, group, 0)

    pltpu.sync_copy(outv, out_hbm.at[pl.ds(base, bpw)])


def kernel(user, item, user_table, item_table, W, b):
    batch = user.shape[0]
    bpw = batch // NW
    ch = bpw // HB
    mesh = plsc.VectorSubcoreMesh(core_axis_name="c", subcore_axis_name="s")

    wb = jnp.broadcast_to(W.reshape(D, 1), (D, L)).reshape(D * L)
    wb = wb.astype(jnp.float32)
    bb = jnp.broadcast_to(b.reshape(1), (L,)).astype(jnp.float32)

    ut4 = _stage(user_table.T)

    k = functools.partial(
        pl.kernel,
        mesh=mesh,
        out_type=jax.ShapeDtypeStruct((batch,), jnp.float32),
        scratch_types=[
            pltpu.VMEM((bpw,), jnp.int32),
            pltpu.VMEM((bpw,), jnp.int32),
            pltpu.VMEM((bpw,), jnp.int32),
            pltpu.VMEM((ch, PW), jnp.float32),
            pltpu.VMEM((ch, D), jnp.float32),
            pltpu.VMEM((bpw,), jnp.float32),
            pltpu.VMEM((D * L,), jnp.float32),
            pltpu.VMEM((L,), jnp.float32),
            pltpu.SemaphoreType.DMA,
            pltpu.SemaphoreType.DMA,
        ],
        compiler_params=pltpu.CompilerParams(
            needs_layout_passes=False, use_tc_tiling_on_sc=False),
    )(functools.partial(_gmf_body, bpw=bpw))

    return k(user.astype(jnp.int32), item.astype(jnp.int32),
             ut4, item_table, wb, bb)


# speedup vs baseline: 11.4375x; 1.0352x over previous
import functools

import jax
import jax.numpy as jnp
from jax import lax
from jax.experimental import pallas as pl
from jax.experimental.pallas import tpu as pltpu
from jax.experimental.pallas import tpu_sc as plsc

D = 32
L = 16
NC = 1
NS = 16
NW = NC * NS
ROWS = 1000000
PK = 4
PW = PK * D
HB = 4

SCW = 4096
SRW = SCW // PK
NP = pl.cdiv(ROWS, SCW)
SR = NP * SRW


def _stage_body(in_ref, out_ref, xs_ref):
    xs_ref[...] = in_ref[...].T
    for s in range(PK):
        out_ref[:, s * D:(s + 1) * D] = xs_ref[s::PK, :]


def _stage(tableT):
    return pl.pallas_call(
        _stage_body,
        grid=(NP,),
        in_specs=[pl.BlockSpec((D, SCW), lambda p: (0, p))],
        out_specs=pl.BlockSpec((SRW, PW), lambda p: (p, 0)),
        out_shape=jax.ShapeDtypeStruct((SR, PW), jnp.float32),
        scratch_shapes=[pltpu.VMEM((SCW, D), jnp.float32)],
    )(tableT)


def _gmf_body(user_hbm, item_hbm, ut_hbm, it_hbm, wb_hbm, bb_hbm, out_hbm,
              uidx, iidx, ublk, iblk, urb, irb, outv, wv, bv, sem_u, sem_i,
              bpw):
    wid = lax.axis_index("s") * NC + lax.axis_index("c")
    base = wid * bpw
    ch = bpw // HB

    pltpu.sync_copy(user_hbm.at[pl.ds(base, bpw)], uidx)
    pltpu.sync_copy(item_hbm.at[pl.ds(base, bpw)], iidx)
    pltpu.sync_copy(wb_hbm, wv)
    pltpu.sync_copy(bb_hbm, bv)

    def blkidx(g, _):
        j0 = g * L
        ublk[pl.ds(j0, L)] = uidx[pl.ds(j0, L)] >> 2
        iblk[pl.ds(j0, L)] = iidx[pl.ds(j0, L)] >> 2
        return 0

    lax.fori_loop(0, bpw // L, blkidx, 0)

    wvecs = [wv[pl.ds(d * L, L)] for d in range(D)]
    bvec = bv[...]
    lane = lax.iota(jnp.int32, L)
    subm = jnp.full((L,), PK - 1, jnp.int32)
    d32 = jnp.full((L,), D, jnp.int32)

    for h in range(HB):
        cp_u = pltpu.async_copy(
            ut_hbm.at[ublk.at[pl.ds(h * ch, ch)]], urb, sem_u)
        cp_i = pltpu.async_copy(
            it_hbm.at[iblk.at[pl.ds(h * ch, ch)]], irb, sem_i)
        cp_u.wait()
        cp_i.wait()

        def group(g, _):
            j0 = h * ch + g * L
            rows = g * L + lane
            cu = (uidx[pl.ds(j0, L)] & subm) * d32
            ci = (iidx[pl.ds(j0, L)] & subm) * d32
            acc = bvec
            for d in range(D):
                cd = jnp.full((L,), d, jnp.int32)
                uc = plsc.load_gather(urb, [rows, cu + cd])
                ic = plsc.load_gather(irb, [rows, ci + cd])
                acc = acc + uc * ic * wvecs[d]
            outv[pl.ds(j0, L)] = acc
            return 0

        lax.fori_loop(0, ch // L, group, 0)

    pltpu.sync_copy(outv, out_hbm.at[pl.ds(base, bpw)])


def kernel(user, item, user_table, item_table, W, b):
    batch = user.shape[0]
    bpw = batch // NW
    ch = bpw // HB
    mesh = plsc.VectorSubcoreMesh(core_axis_name="c", subcore_axis_name="s",
                                  num_cores=NC)

    wb = jnp.broadcast_to(W.reshape(D, 1), (D, L)).reshape(D * L)
    wb = wb.astype(jnp.float32)
    bb = jnp.broadcast_to(b.reshape(1), (L,)).astype(jnp.float32)

    ut4 = _stage(user_table.T)
    it4 = _stage(item_table.T)

    k = functools.partial(
        pl.kernel,
        mesh=mesh,
        out_type=jax.ShapeDtypeStruct((batch,), jnp.float32),
        scratch_types=[
            pltpu.VMEM((bpw,), jnp.int32),
            pltpu.VMEM((bpw,), jnp.int32),
            pltpu.VMEM((bpw,), jnp.int32),
            pltpu.VMEM((bpw,), jnp.int32),
            pltpu.VMEM((ch, PW), jnp.float32),
            pltpu.VMEM((ch, PW), jnp.float32),
            pltpu.VMEM((bpw,), jnp.float32),
            pltpu.VMEM((D * L,), jnp.float32),
            pltpu.VMEM((L,), jnp.float32),
            pltpu.SemaphoreType.DMA,
            pltpu.SemaphoreType.DMA,
        ],
        compiler_params=pltpu.CompilerParams(
            needs_layout_passes=False, use_tc_tiling_on_sc=False),
    )(functools.partial(_gmf_body, bpw=bpw))

    return k(user.astype(jnp.int32), item.astype(jnp.int32),
             ut4, it4, wb, bb)


# speedup vs baseline: 11.8021x; 1.0319x over previous
import functools

import jax
import jax.numpy as jnp
from jax import lax
from jax.experimental import pallas as pl
from jax.experimental.pallas import tpu as pltpu
from jax.experimental.pallas import tpu_sc as plsc

D = 32
L = 16
NC = 2
NS = 16
NW = NC * NS
ROWS = 1000000
PK = 4
PW = PK * D
HB = 2

SCW = 4096
SRW = SCW // PK
NP = pl.cdiv(ROWS, SCW)
SR = NP * SRW


def _stage_body(in_ref, out_ref, xs_ref):
    xs_ref[...] = in_ref[...].T
    for s in range(PK):
        out_ref[:, s * D:(s + 1) * D] = xs_ref[s::PK, :]


def _stage(tableT):
    return pl.pallas_call(
        _stage_body,
        grid=(NP,),
        in_specs=[pl.BlockSpec((D, SCW), lambda p: (0, p))],
        out_specs=pl.BlockSpec((SRW, PW), lambda p: (p, 0)),
        out_shape=jax.ShapeDtypeStruct((SR, PW), jnp.float32),
        scratch_shapes=[pltpu.VMEM((SCW, D), jnp.float32)],
    )(tableT)


def _gmf_body(user_hbm, item_hbm, ut_hbm, it_hbm, wb_hbm, bb_hbm, out_hbm,
              uidx, iidx, ublk, iblk, urb, irb, outv, wv, bv, sem_u, sem_i,
              bpw):
    wid = lax.axis_index("s") * NC + lax.axis_index("c")
    base = wid * bpw
    ch = bpw // HB

    pltpu.sync_copy(user_hbm.at[pl.ds(base, bpw)], uidx)
    pltpu.sync_copy(item_hbm.at[pl.ds(base, bpw)], iidx)
    pltpu.sync_copy(wb_hbm, wv)
    pltpu.sync_copy(bb_hbm, bv)

    def blkidx(g, _):
        j0 = g * L
        ublk[pl.ds(j0, L)] = uidx[pl.ds(j0, L)] >> 2
        iblk[pl.ds(j0, L)] = iidx[pl.ds(j0, L)] >> 2
        return 0

    lax.fori_loop(0, bpw // L, blkidx, 0)

    wvecs = [wv[pl.ds(d * L, L)] for d in range(D)]
    bvec = bv[...]
    lane = lax.iota(jnp.int32, L)
    subm = jnp.full((L,), PK - 1, jnp.int32)
    d32 = jnp.full((L,), D, jnp.int32)

    for h in range(HB):
        cp_u = pltpu.async_copy(
            ut_hbm.at[ublk.at[pl.ds(h * ch, ch)]], urb, sem_u)
        cp_i = pltpu.async_copy(
            it_hbm.at[iblk.at[pl.ds(h * ch, ch)]], irb, sem_i)
        cp_u.wait()
        cp_i.wait()

        def group(g, _):
            j0 = h * ch + g * L
            rows = g * L + lane
            cu = (uidx[pl.ds(j0, L)] & subm) * d32
            ci = (iidx[pl.ds(j0, L)] & subm) * d32
            acc = bvec
            for d in range(D):
                cd = jnp.full((L,), d, jnp.int32)
                uc = plsc.load_gather(urb, [rows, cu + cd])
                ic = plsc.load_gather(irb, [rows, ci + cd])
                acc = acc + uc * ic * wvecs[d]
            outv[pl.ds(j0, L)] = acc
            return 0

        lax.fori_loop(0, ch // L, group, 0)

    pltpu.sync_copy(outv, out_hbm.at[pl.ds(base, bpw)])


def kernel(user, item, user_table, item_table, W, b):
    batch = user.shape[0]
    bpw = batch // NW
    ch = bpw // HB
    mesh = plsc.VectorSubcoreMesh(core_axis_name="c", subcore_axis_name="s",
                                  num_cores=NC)

    wb = jnp.broadcast_to(W.reshape(D, 1), (D, L)).reshape(D * L)
    wb = wb.astype(jnp.float32)
    bb = jnp.broadcast_to(b.reshape(1), (L,)).astype(jnp.float32)

    ut4 = _stage(user_table.T)
    it4 = _stage(item_table.T)

    k = functools.partial(
        pl.kernel,
        mesh=mesh,
        out_type=jax.ShapeDtypeStruct((batch,), jnp.float32),
        scratch_types=[
            pltpu.VMEM((bpw,), jnp.int32),
            pltpu.VMEM((bpw,), jnp.int32),
            pltpu.VMEM((bpw,), jnp.int32),
            pltpu.VMEM((bpw,), jnp.int32),
            pltpu.VMEM((ch, PW), jnp.float32),
            pltpu.VMEM((ch, PW), jnp.float32),
            pltpu.VMEM((bpw,), jnp.float32),
            pltpu.VMEM((D * L,), jnp.float32),
            pltpu.VMEM((L,), jnp.float32),
            pltpu.SemaphoreType.DMA,
            pltpu.SemaphoreType.DMA,
        ],
        compiler_params=pltpu.CompilerParams(
            needs_layout_passes=False, use_tc_tiling_on_sc=False),
    )(functools.partial(_gmf_body, bpw=bpw))

    return k(user.astype(jnp.int32), item.astype(jnp.int32),
             ut4, it4, wb, bb)


# speedup vs baseline: 12.9770x; 1.0996x over previous
import functools

import jax
import jax.numpy as jnp
from jax import lax
from jax.experimental import pallas as pl
from jax.experimental.pallas import tpu as pltpu
from jax.experimental.pallas import tpu_sc as plsc

D = 32
L = 16
NC = 2
NS = 16
NW = NC * NS
ROWS = 1000000
PK = 4
PW = PK * D
HB = 2

SCW = 4096
SRW = SCW // PK
NP = pl.cdiv(ROWS, SCW)
SR = NP * SRW


def _stage_body(inu_ref, ini_ref, outu_ref, outi_ref, xsu_ref, xsi_ref):
    xsu_ref[...] = inu_ref[...].T
    xsi_ref[...] = ini_ref[...].T
    for s in range(PK):
        outu_ref[:, s * D:(s + 1) * D] = xsu_ref[s::PK, :]
        outi_ref[:, s * D:(s + 1) * D] = xsi_ref[s::PK, :]


def _stage(utT, itT):
    sds = jax.ShapeDtypeStruct((SR, PW), jnp.float32)
    return pl.pallas_call(
        _stage_body,
        grid=(NP,),
        in_specs=[pl.BlockSpec((D, SCW), lambda p: (0, p)),
                  pl.BlockSpec((D, SCW), lambda p: (0, p))],
        out_specs=[pl.BlockSpec((SRW, PW), lambda p: (p, 0)),
                   pl.BlockSpec((SRW, PW), lambda p: (p, 0))],
        out_shape=[sds, sds],
        scratch_shapes=[pltpu.VMEM((SCW, D), jnp.float32),
                        pltpu.VMEM((SCW, D), jnp.float32)],
    )(utT, itT)


def _gmf_body(user_hbm, item_hbm, ut_hbm, it_hbm, wb_hbm, bb_hbm, out_hbm,
              uidx, iidx, ublk, iblk, urb, irb, outv, wv, bv, sem_u, sem_i,
              bpw):
    wid = lax.axis_index("s") * NC + lax.axis_index("c")
    base = wid * bpw
    ch = bpw // HB

    pltpu.sync_copy(user_hbm.at[pl.ds(base, bpw)], uidx)
    pltpu.sync_copy(item_hbm.at[pl.ds(base, bpw)], iidx)
    pltpu.sync_copy(wb_hbm, wv)
    pltpu.sync_copy(bb_hbm, bv)

    def blkidx(g, _):
        j0 = g * L
        ublk[pl.ds(j0, L)] = uidx[pl.ds(j0, L)] >> 2
        iblk[pl.ds(j0, L)] = iidx[pl.ds(j0, L)] >> 2
        return 0

    lax.fori_loop(0, bpw // L, blkidx, 0)

    wvecs = [wv[pl.ds(d * L, L)] for d in range(D)]
    bvec = bv[...]
    lane = lax.iota(jnp.int32, L)
    subm = jnp.full((L,), PK - 1, jnp.int32)
    d32 = jnp.full((L,), D, jnp.int32)

    for h in range(HB):
        cp_u = pltpu.async_copy(
            ut_hbm.at[ublk.at[pl.ds(h * ch, ch)]], urb, sem_u)
        cp_i = pltpu.async_copy(
            it_hbm.at[iblk.at[pl.ds(h * ch, ch)]], irb, sem_i)
        cp_u.wait()
        cp_i.wait()

        def group(g, _):
            j0 = h * ch + g * L
            rows = g * L + lane
            cu = (uidx[pl.ds(j0, L)] & subm) * d32
            ci = (iidx[pl.ds(j0, L)] & subm) * d32
            acc = bvec
            for d in range(D):
                cd = jnp.full((L,), d, jnp.int32)
                uc = plsc.load_gather(urb, [rows, cu + cd])
                ic = plsc.load_gather(irb, [rows, ci + cd])
                acc = acc + uc * ic * wvecs[d]
            outv[pl.ds(j0, L)] = acc
            return 0

        lax.fori_loop(0, ch // L, group, 0)

    pltpu.sync_copy(outv, out_hbm.at[pl.ds(base, bpw)])


def kernel(user, item, user_table, item_table, W, b):
    batch = user.shape[0]
    bpw = batch // NW
    ch = bpw // HB
    mesh = plsc.VectorSubcoreMesh(core_axis_name="c", subcore_axis_name="s",
                                  num_cores=NC)

    wb = jnp.broadcast_to(W.reshape(D, 1), (D, L)).reshape(D * L)
    wb = wb.astype(jnp.float32)
    bb = jnp.broadcast_to(b.reshape(1), (L,)).astype(jnp.float32)

    ut4, it4 = _stage(user_table.T, item_table.T)

    k = functools.partial(
        pl.kernel,
        mesh=mesh,
        out_type=jax.ShapeDtypeStruct((batch,), jnp.float32),
        scratch_types=[
            pltpu.VMEM((bpw,), jnp.int32),
            pltpu.VMEM((bpw,), jnp.int32),
            pltpu.VMEM((bpw,), jnp.int32),
            pltpu.VMEM((bpw,), jnp.int32),
            pltpu.VMEM((ch, PW), jnp.float32),
            pltpu.VMEM((ch, PW), jnp.float32),
            pltpu.VMEM((bpw,), jnp.float32),
            pltpu.VMEM((D * L,), jnp.float32),
            pltpu.VMEM((L,), jnp.float32),
            pltpu.SemaphoreType.DMA,
            pltpu.SemaphoreType.DMA,
        ],
        compiler_params=pltpu.CompilerParams(
            needs_layout_passes=False, use_tc_tiling_on_sc=False),
    )(functools.partial(_gmf_body, bpw=bpw))

    return k(user.astype(jnp.int32), item.astype(jnp.int32),
             ut4, it4, wb, bb)
